# baseline (speedup 1.0000x reference)
import functools

import jax
import jax.numpy as jnp
from jax import lax
from jax.experimental import pallas as pl
from jax.experimental.pallas import tpu as pltpu
from jax.experimental.pallas import tpu_sc as plsc

_NC = 2
_NS = 16
_NW = _NC * _NS
_LANES = 16
_BATCH = 16384
_DIM = 128
_BPW = _BATCH // _NW
_CHUNK = 64
_NCHUNK = _BPW // _CHUNK
_NBUF = 5
_VECS = _DIM // _LANES


def _sc_body(ind_hbm, eeg_hbm, ir_hbm, b_hbm, out_hbm,
             idx_v, rows_v, eeg_v, ir_v, acc_v, sems):
    c = lax.axis_index("c")
    s = lax.axis_index("s")
    wid = c * _NS + s
    base = wid * _BPW

    pltpu.sync_copy(ind_hbm.at[pl.ds(wid * 4, 4)], idx_v)

    def fire(ch):
        buf = ch % _NBUF
        row0 = base + ch * _CHUNK
        return (
            pltpu.async_copy(
                b_hbm.at[idx_v.at[ch // 2, pl.ds((ch % 2) * _CHUNK, _CHUNK)]],
                rows_v.at[buf], sems.at[buf, 0]),
            pltpu.async_copy(eeg_hbm.at[pl.ds(row0, _CHUNK)], eeg_v.at[buf],
                             sems.at[buf, 1]),
            pltpu.async_copy(ir_hbm.at[pl.ds(row0, _CHUNK)], ir_v.at[buf],
                             sems.at[buf, 2]),
        )

    zero = jnp.zeros((_LANES,), jnp.float32)
    accs = (zero, zero, zero, zero)

    inflight = [fire(0), fire(1), fire(2), fire(3)]
    for ch in range(_NCHUNK):
        buf = ch % _NBUF
        for cp in inflight[0]:
            cp.wait()
        inflight = inflight[1:]
        if ch + 4 < _NCHUNK:
            inflight.append(fire(ch + 4))

        @plsc.parallel_loop(0, _CHUNK, unroll=4, carry=accs)
        def _row(r, carry):
            a = list(carry)
            for j in range(_VECS):
                col = j * _LANES
                b = rows_v[buf, r, pl.ds(col, _LANES)]
                e = eeg_v[buf, r, pl.ds(col, _LANES)]
                i = ir_v[buf, r, pl.ds(col, _LANES)]
                de = b - e
                di = b - i
                a[j % 2] = a[j % 2] + de * de
                a[2 + j % 2] = a[2 + j % 2] + di * di
            return tuple(a)

        accs = _row

    acc_v[...] = (accs[0] + accs[1]) + (accs[2] + accs[3])
    pltpu.sync_copy(acc_v, out_hbm.at[wid])


@jax.jit
def _quant_loss(ind2, eeg, ir, b):
    mesh = plsc.VectorSubcoreMesh(
        core_axis_name="c", subcore_axis_name="s",
        num_cores=_NC, num_subcores=_NS)
    partials = pl.kernel(
        _sc_body,
        out_type=jax.ShapeDtypeStruct((_NW, _LANES), jnp.float32),
        mesh=mesh,
        scratch_types=[
            pltpu.VMEM((4, 128), jnp.int32),
            pltpu.VMEM((_NBUF, _CHUNK, _DIM), jnp.float32),
            pltpu.VMEM((_NBUF, _CHUNK, _DIM), jnp.float32),
            pltpu.VMEM((_NBUF, _CHUNK, _DIM), jnp.float32),
            pltpu.VMEM((_LANES,), jnp.float32),
            pltpu.SemaphoreType.DMA((_NBUF, 3)),
        ],
    )(ind2, eeg, ir, b)
    return jnp.sum(partials)


def kernel(eeg, ir, ind, B, un_eeg, un_ir, device):
    ind2 = ind.astype(jnp.int32).reshape(128, 128)
    return _quant_loss(ind2, eeg, ir, B)


# speedup vs baseline: 1.0285x; 1.0285x over previous
import functools

import jax
import jax.numpy as jnp
from jax import lax
from jax.experimental import pallas as pl
from jax.experimental.pallas import tpu as pltpu
from jax.experimental.pallas import tpu_sc as plsc

_NC = 2
_NS = 16
_NW = _NC * _NS
_LANES = 16
_BATCH = 16384
_DIM = 128
_BPW = _BATCH // _NW
_CHUNK = 64
_NCHUNK = _BPW // _CHUNK
_NBUF = 4
_VECS = _DIM // _LANES


def _sc_body(ind_hbm, eeg_hbm, ir_hbm, b_hbm, out_hbm,
             idx_v, rows_v, eeg_v, ir_v, acc_v, sems):
    c = lax.axis_index("c")
    s = lax.axis_index("s")
    wid = c * _NS + s
    base = wid * _BPW

    pltpu.sync_copy(ind_hbm.at[pl.ds(wid * 4, 4)], idx_v)

    def fire(ch):
        buf = ch % _NBUF
        row0 = base + ch * _CHUNK
        return (
            pltpu.async_copy(
                b_hbm.at[idx_v.at[ch // 2, pl.ds((ch % 2) * _CHUNK, _CHUNK)]],
                rows_v.at[buf], sems.at[buf, 0]),
            pltpu.async_copy(eeg_hbm.at[pl.ds(row0, _CHUNK)], eeg_v.at[buf],
                             sems.at[buf, 1]),
            pltpu.async_copy(ir_hbm.at[pl.ds(row0, _CHUNK)], ir_v.at[buf],
                             sems.at[buf, 2]),
        )

    zero = jnp.zeros((_LANES,), jnp.float32)
    accs = (zero, zero, zero, zero)

    inflight = [fire(0), fire(1), fire(2)]
    for ch in range(_NCHUNK):
        buf = ch % _NBUF
        for cp in inflight[0]:
            cp.wait()
        inflight = inflight[1:]
        if ch + 3 < _NCHUNK:
            inflight.append(fire(ch + 3))

        @plsc.parallel_loop(0, _CHUNK, unroll=2, carry=accs)
        def _row(r, carry):
            a = list(carry)
            for j in range(_VECS):
                col = j * _LANES
                b = rows_v[buf, r, pl.ds(col, _LANES)]
                e = eeg_v[buf, r, pl.ds(col, _LANES)]
                i = ir_v[buf, r, pl.ds(col, _LANES)]
                de = b - e
                di = b - i
                a[j % 2] = a[j % 2] + de * de
                a[2 + j % 2] = a[2 + j % 2] + di * di
            return tuple(a)

        accs = _row

    acc_v[...] = (accs[0] + accs[1]) + (accs[2] + accs[3])
    pltpu.sync_copy(acc_v, out_hbm.at[wid])


@jax.jit
def _quant_loss(ind2, eeg, ir, b):
    mesh = plsc.VectorSubcoreMesh(
        core_axis_name="c", subcore_axis_name="s",
        num_cores=_NC, num_subcores=_NS)
    partials = pl.kernel(
        _sc_body,
        out_type=jax.ShapeDtypeStruct((_NW, _LANES), jnp.float32),
        mesh=mesh,
        scratch_types=[
            pltpu.VMEM((4, 128), jnp.int32),
            pltpu.VMEM((_NBUF, _CHUNK, _DIM), jnp.float32),
            pltpu.VMEM((_NBUF, _CHUNK, _DIM), jnp.float32),
            pltpu.VMEM((_NBUF, _CHUNK, _DIM), jnp.float32),
            pltpu.VMEM((_LANES,), jnp.float32),
            pltpu.SemaphoreType.DMA((_NBUF, 3)),
        ],
    )(ind2, eeg, ir, b)
    return jnp.sum(partials)


def kernel(eeg, ir, ind, B, un_eeg, un_ir, device):
    ind2 = ind.astype(jnp.int32).reshape(128, 128)
    return _quant_loss(ind2, eeg, ir, B)
